# Initial kernel scaffold; baseline (speedup 1.0000x reference)
import jax
import jax.numpy as jnp
from jax.experimental import pallas as pl


def kernel(X, W1_0, b1_0, W1_1, b1_1, W2_0, b2_0, W2_1, b2_1):
    raise NotImplementedError("write your pallas kernel here")


# speedup vs baseline: 1.4288x; 1.4288x over previous
import jax
import jax.numpy as jnp
from jax.experimental import pallas as pl
from jax.experimental.pallas import tpu as pltpu

B, N, EMB = 16, 64, 128
IB = 2


def _fwl_kernel(x_ref, w0_ref, b0_ref, wd_ref, bd_ref, o_ref, x1_ref):
    x = x_ref[0].reshape(N * N, EMB)
    h = jnp.dot(x.astype(jnp.bfloat16), w0_ref[...],
                preferred_element_type=jnp.float32)
    h = jnp.maximum(h + b0_ref[...], 0.0)
    x12 = jnp.dot(h.astype(jnp.bfloat16), wd_ref[...],
                  preferred_element_type=jnp.float32)
    x12 = jnp.maximum(x12 + bd_ref[...], 0.0)
    x1_ref[...] = x12[:, :EMB].reshape(N, N, EMB)
    x2 = x12[:, EMB:].reshape(N, N, EMB)

    def iblock(ib, carry):
        a = x1_ref[pl.ds(ib * IB, IB)]
        acc = jnp.zeros((IB, N, EMB), jnp.float32)
        for k in range(N):
            acc = acc + a[:, k, None, :] * x2[k]
        o_ref[0, pl.ds(ib * IB, IB)] = acc
        return carry

    jax.lax.fori_loop(0, N // IB, iblock, 0)


def kernel(X, W1_0, b1_0, W1_1, b1_1, W2_0, b2_0, W2_1, b2_1):
    f32 = jnp.float32
    bf16 = jnp.bfloat16
    w0 = jnp.concatenate([W1_0, W2_0], axis=1).astype(bf16)
    z = jnp.zeros((EMB, EMB), f32)
    wd = jnp.concatenate(
        [jnp.concatenate([W1_1, z], axis=1),
         jnp.concatenate([z, W2_1], axis=1)], axis=0).astype(bf16)
    b0 = jnp.concatenate([b1_0, b2_0]).reshape(1, 2 * EMB).astype(f32)
    bd = jnp.concatenate([b1_1, b2_1]).reshape(1, 2 * EMB).astype(f32)

    return pl.pallas_call(
        _fwl_kernel,
        grid=(B,),
        in_specs=[
            pl.BlockSpec((1, N, N, EMB), lambda b: (b, 0, 0, 0)),
            pl.BlockSpec((EMB, 2 * EMB), lambda b: (0, 0)),
            pl.BlockSpec((1, 2 * EMB), lambda b: (0, 0)),
            pl.BlockSpec((2 * EMB, 2 * EMB), lambda b: (0, 0)),
            pl.BlockSpec((1, 2 * EMB), lambda b: (0, 0)),
        ],
        out_specs=pl.BlockSpec((1, N, N, EMB), lambda b: (b, 0, 0, 0)),
        out_shape=jax.ShapeDtypeStruct((B, N, N, EMB), f32),
        scratch_shapes=[pltpu.VMEM((N, N, EMB), f32)],
    )(X, w0, b0, wd, bd)
